# Initial kernel scaffold; baseline (speedup 1.0000x reference)
import jax
import jax.numpy as jnp
from jax.experimental import pallas as pl


def kernel(all_points, idx0, idx1, idx2, sa1_params, sa2_params, fin_params, fc_params):
    raise NotImplementedError("write your pallas kernel here")


# speedup vs baseline: 3.0592x; 3.0592x over previous
import jax
import jax.numpy as jnp
from jax import lax
from jax.experimental import pallas as pl

_K = 64
_INF = float('inf')


def _dot(a, b, prec=lax.Precision.HIGHEST):
    return lax.dot_general(a, b, (((1,), (0,)), ((), ())),
                           precision=prec, preferred_element_type=jnp.float32)


def _sqdist(q, pt):
    d0 = (q[:, 0:1] - pt[0:1, :]) ** 2
    d1 = (q[:, 1:2] - pt[1:2, :]) ** 2
    d2 = (q[:, 2:3] - pt[2:3, :]) ** 2
    return (d0 + d1) + d2


def _knn_mlp_max(D, P1e, Q1e, W2t, b2, W3t, b3):
    M, P = D.shape
    iot = lax.broadcasted_iota(jnp.int32, (M, P), 1)
    big = jnp.int32(2 ** 30)
    acc0 = jnp.full((M, W3t.shape[1]), -_INF, jnp.float32)

    def step(_, carry):
        D, acc = carry
        m = jnp.min(D, axis=1, keepdims=True)
        col = jnp.min(jnp.where(D == m, iot, big), axis=1, keepdims=True)
        sel = iot == col
        h = _dot(sel.astype(jnp.float32), P1e) + Q1e
        h = jnp.maximum(h, 0.0)
        h = jnp.maximum(_dot(h, W2t, prec=None) + b2, 0.0)
        h = jnp.maximum(_dot(h, W3t, prec=None) + b3, 0.0)
        return jnp.where(sel, _INF, D), jnp.maximum(acc, h)

    _, acc = lax.fori_loop(0, _K, step, (D, acc0))
    return acc


def _sa1_body(ap_ref, apt_ref, i0c_ref, i0r_ref, i1c_ref, i1r_ref,
              a1t_ref, q1w_ref, b1v_ref, w2t_ref, b2v_ref, w3t_ref, b3v_ref,
              pos1_ref, pos1t_ref, feat1_ref):
    ap = ap_ref[0]
    apt = apt_ref[0]
    i0c = i0c_ref[0]
    i0r = i0r_ref[0]
    i1c = i1c_ref[0]
    i1r = i1r_ref[0]
    N, M0, M1 = ap.shape[0], i0c.shape[0], i1c.shape[0]

    CH = min(N, 1024)
    pos0 = jnp.zeros((M0, 3), jnp.float32)
    pos0t = jnp.zeros((3, M0), jnp.float32)
    for s in range(0, N, CH):
        ii = lax.broadcasted_iota(jnp.int32, (M0, CH), 1) + s
        pos0 = pos0 + _dot((i0c == ii).astype(jnp.float32), ap[s:s + CH])
        jj = lax.broadcasted_iota(jnp.int32, (CH, M0), 0) + s
        pos0t = pos0t + _dot(apt[:, s:s + CH], (jj == i0r).astype(jnp.float32))

    oh1 = (i1c == lax.broadcasted_iota(jnp.int32, (M1, M0), 1)).astype(jnp.float32)
    q1 = _dot(oh1, pos0)
    oh1t = (lax.broadcasted_iota(jnp.int32, (M0, M1), 0) == i1r).astype(jnp.float32)
    q1t = _dot(pos0t, oh1t)

    P1e = _dot(pos0, a1t_ref[...], prec=None)
    Q1e = _dot(q1, q1w_ref[...], prec=None) + b1v_ref[...]
    D = _sqdist(q1, pos0t)

    acc = _knn_mlp_max(D, P1e, Q1e, w2t_ref[...], b2v_ref[...],
                       w3t_ref[...], b3v_ref[...])
    pos1_ref[0] = q1
    pos1t_ref[0] = q1t
    feat1_ref[0] = acc


def _sa2_body(p1_ref, p1t_ref, f1_ref, i2c_ref,
              wp2t_ref, wf2t_ref, q2w_ref, b1v_ref, w2t_ref, b2v_ref,
              w3t_ref, b3v_ref, pos2_ref, feat2_ref):
    pos1 = p1_ref[0]
    pos1t = p1t_ref[0]
    feat1 = f1_ref[0]
    i2c = i2c_ref[0]
    M1, M2 = pos1.shape[0], i2c.shape[0]

    oh2 = (i2c == lax.broadcasted_iota(jnp.int32, (M2, M1), 1)).astype(jnp.float32)
    q2 = _dot(oh2, pos1)
    P2e = _dot(feat1, wf2t_ref[...], prec=None) + _dot(pos1, wp2t_ref[...], prec=None)
    Q2e = _dot(q2, q2w_ref[...], prec=None) + b1v_ref[...]
    D = _sqdist(q2, pos1t)

    acc = _knn_mlp_max(D, P2e, Q2e, w2t_ref[...], b2v_ref[...],
                       w3t_ref[...], b3v_ref[...])
    pos2_ref[0] = q2
    feat2_ref[0] = acc


def _fin_body(p2_ref, f2_ref, fp1_ref, ff1_ref, fb1_ref, f2w_ref, fb2_ref,
              f3w_ref, fb3_ref, c1w_ref, c1b_ref, c2w_ref, c2b_ref,
              c3w_ref, c3b_ref, out_ref):
    pos2 = p2_ref[0]
    feat2 = f2_ref[0]
    x = _dot(pos2, fp1_ref[...], prec=None) + _dot(feat2, ff1_ref[...], prec=None)
    x = jnp.maximum(x + fb1_ref[...], 0.0)
    x = jnp.maximum(_dot(x, f2w_ref[...], prec=None) + fb2_ref[...], 0.0)
    x = jnp.maximum(_dot(x, f3w_ref[...], prec=None) + fb3_ref[...], 0.0)
    pooled = jnp.max(x, axis=0, keepdims=True)
    y = jnp.maximum(_dot(pooled, c1w_ref[...], prec=None) + c1b_ref[...], 0.0)
    y = jnp.maximum(_dot(y, c2w_ref[...], prec=None) + c2b_ref[...], 0.0)
    out_ref[0] = _dot(y, c3w_ref[...], prec=None) + c3b_ref[...]


def _full(shape):
    return pl.BlockSpec(shape, lambda b: (0,) * len(shape))


def _batched(shape):
    return pl.BlockSpec((1,) + shape, lambda b: (b,) + (0,) * len(shape))


def kernel(all_points, idx0, idx1, idx2, sa1_params, sa2_params, fin_params, fc_params):
    B, N, _ = all_points.shape
    M0, M1, M2 = idx0.shape[1], idx1.shape[1], idx2.shape[1]
    f32 = jnp.float32

    ap = all_points.astype(f32)
    apt = jnp.transpose(ap, (0, 2, 1))
    i0 = idx0.astype(jnp.int32)
    i1 = idx1.astype(jnp.int32)
    i2 = idx2.astype(jnp.int32)
    i0c, i0r = i0[:, :, None], i0[:, None, :]
    i1c, i1r = i1[:, :, None], i1[:, None, :]
    i2c = i2[:, :, None]

    (W1, g1, b1), (W2, g2, b2), (W3, g3, b3) = sa1_params
    a1t = ((W1[:, :3] + W1[:, 3:]) * g1[:, None]).T
    q1w = (-(W1[:, :3]) * g1[:, None]).T
    s1 = dict(a1t=a1t, q1w=q1w, b1v=b1[None, :],
              w2t=(W2 * g2[:, None]).T, b2v=b2[None, :],
              w3t=(W3 * g3[:, None]).T, b3v=b3[None, :])

    (V1, h1, c1), (V2, h2, c2), (V3, h3, c3) = sa2_params
    wp2t = (V1[:, :3] * h1[:, None]).T
    wf2t = (V1[:, 3:] * h1[:, None]).T
    s2 = dict(wp2t=wp2t, wf2t=wf2t, q2w=-wp2t, b1v=c1[None, :],
              w2t=(V2 * h2[:, None]).T, b2v=c2[None, :],
              w3t=(V3 * h3[:, None]).T, b3v=c3[None, :])

    (U1, e1, d1), (U2, e2, d2), (U3, e3, d3) = fin_params
    fin = dict(fp1=(U1[:, :3] * e1[:, None]).T, ff1=(U1[:, 3:] * e1[:, None]).T,
               fb1=d1[None, :],
               f2w=(U2 * e2[:, None]).T, fb2=d2[None, :],
               f3w=(U3 * e3[:, None]).T, fb3=d3[None, :])
    F1, fg1, fb1, F2, fg2, fb2, F3, fb3 = fc_params
    fc = dict(c1w=(F1 * fg1[:, None]).T, c1b=fb1[None, :],
              c2w=(F2 * fg2[:, None]).T, c2b=fb2[None, :],
              c3w=F3.T, c3b=fb3[None, :])

    C1 = s1['w3t'].shape[1]
    C2 = s2['w3t'].shape[1]

    s1_keys = ['a1t', 'q1w', 'b1v', 'w2t', 'b2v', 'w3t', 'b3v']
    pos1, pos1t, feat1 = pl.pallas_call(
        _sa1_body,
        grid=(B,),
        in_specs=[_batched((N, 3)), _batched((3, N)),
                  _batched((M0, 1)), _batched((1, M0)),
                  _batched((M1, 1)), _batched((1, M1))] +
                 [_full(s1[k].shape) for k in s1_keys],
        out_specs=[_batched((M1, 3)), _batched((3, M1)), _batched((M1, C1))],
        out_shape=[jax.ShapeDtypeStruct((B, M1, 3), f32),
                   jax.ShapeDtypeStruct((B, 3, M1), f32),
                   jax.ShapeDtypeStruct((B, M1, C1), f32)],
    )(ap, apt, i0c, i0r, i1c, i1r, *[s1[k] for k in s1_keys])

    s2_keys = ['wp2t', 'wf2t', 'q2w', 'b1v', 'w2t', 'b2v', 'w3t', 'b3v']
    pos2, feat2 = pl.pallas_call(
        _sa2_body,
        grid=(B,),
        in_specs=[_batched((M1, 3)), _batched((3, M1)), _batched((M1, C1)),
                  _batched((M2, 1))] +
                 [_full(s2[k].shape) for k in s2_keys],
        out_specs=[_batched((M2, 3)), _batched((M2, C2))],
        out_shape=[jax.ShapeDtypeStruct((B, M2, 3), f32),
                   jax.ShapeDtypeStruct((B, M2, C2), f32)],
    )(pos1, pos1t, feat1, i2c, *[s2[k] for k in s2_keys])

    fin_keys = ['fp1', 'ff1', 'fb1', 'f2w', 'fb2', 'f3w', 'fb3']
    fc_keys = ['c1w', 'c1b', 'c2w', 'c2b', 'c3w', 'c3b']
    out = pl.pallas_call(
        _fin_body,
        grid=(B,),
        in_specs=[_batched((M2, 3)), _batched((M2, C2))] +
                 [_full(fin[k].shape) for k in fin_keys] +
                 [_full(fc[k].shape) for k in fc_keys],
        out_specs=[_batched((1, fc['c3w'].shape[1]))],
        out_shape=[jax.ShapeDtypeStruct((B, 1, fc['c3w'].shape[1]), f32)],
    )(pos2, feat2, *[fin[k] for k in fin_keys], *[fc[k] for k in fc_keys])[0]

    return out[:, 0, :]
